# baseline (speedup 1.0000x reference)
import jax
import jax.numpy as jnp
from jax.experimental import pallas as pl
from jax.experimental.pallas import tpu as pltpu

BB, MM = 4, 4096
N1, N2 = 1024, 256
KNN = 64
R1SQ = 0.2 * 0.2
R2SQ = 0.4 * 0.4
NEG = -1e30
QT1 = 256


def _local_body(pos_ref, wl0_ref, bl0_ref, wl1_ref, bl1_ref, out_ref):
    px = pos_ref[0, :, 0:1]
    py = pos_ref[0, :, 1:2]
    h = jnp.tanh(px * wl0_ref[0:1, :] + py * wl0_ref[1:2, :] + bl0_ref[...])
    out_ref[0] = jnp.tanh(
        jax.lax.dot_general(h, wl1_ref[...], (((1,), (0,)), ((), ())),
                            preferred_element_type=jnp.float32)
        + bl1_ref[...])


def _fps_run(px, py, n):
    b, msrc = px.shape
    iota = jax.lax.broadcasted_iota(jnp.int32, (b, msrc), 1)
    slot_iota = jax.lax.broadcasted_iota(jnp.int32, (b, n), 1)

    qx0 = jnp.where(slot_iota == 0, px[:, 0:1], 0.0)
    qy0 = jnp.where(slot_iota == 0, py[:, 0:1], 0.0)
    mind0 = jnp.full((b, msrc), jnp.inf, dtype=jnp.float32)

    def body(i, st):
        qx, qy, mind, lx, ly = st
        d = (px - lx) ** 2 + (py - ly) ** 2
        mind = jnp.minimum(mind, d)
        mx = jnp.max(mind, axis=1, keepdims=True)
        isel = jnp.min(jnp.where(mind == mx, iota, msrc), axis=1, keepdims=True)
        sel = iota == isel
        nlx = jnp.sum(jnp.where(sel, px, 0.0), axis=1, keepdims=True)
        nly = jnp.sum(jnp.where(sel, py, 0.0), axis=1, keepdims=True)
        slot = slot_iota == (i + 1)
        qx = jnp.where(slot, nlx, qx)
        qy = jnp.where(slot, nly, qy)
        return (qx, qy, mind, nlx, nly)

    qx, qy, _, _, _ = jax.lax.fori_loop(
        0, n - 1, body, (qx0, qy0, mind0, px[:, 0:1], py[:, 0:1]))
    return qx, qy


def _fps_body(pxy_ref, q1_ref, q2_ref):
    px = pxy_ref[0]
    py = pxy_ref[1]
    q1x, q1y = _fps_run(px, py, N1)
    q1_ref[0] = q1x
    q1_ref[1] = q1y
    q2x, q2y = _fps_run(q1x, q1y, N2)
    q2_ref[0] = q2x
    q2_ref[1] = q2y


def _knn_conv(qx, qy, px, py, p_feat, cq, w2, b2, rsq, out_ref, dm_ref,
              acc_ref):
    q, _ = qx.shape
    msrc = px.shape[1]
    inf = jnp.float32(jnp.inf)
    c = w2.shape[1]
    f = w2.shape[0]
    iota = jax.lax.broadcasted_iota(jnp.int32, (q, msrc), 1)
    paug = jnp.concatenate(
        [p_feat, jnp.ones((msrc, 1), jnp.float32)], axis=1)

    d = (qx - px) ** 2 + (qy - py) ** 2
    dm_ref[...] = jnp.where(d <= rsq, d, inf)
    acc_ref[...] = jnp.full((q, c), NEG, jnp.float32)
    cq2 = jnp.concatenate([cq, cq], axis=0)

    def exact_extract(dm):
        m = jnp.min(dm, axis=1, keepdims=True)
        cand = jnp.where(dm == m, iota, msrc)
        isel = jnp.min(cand, axis=1, keepdims=True)
        oh = cand == isel
        return m, oh.astype(jnp.float32), jnp.where(oh, inf, dm)

    def pass_body(_, carry):
        dm = dm_ref[...]
        m1 = jnp.min(dm, axis=1, keepdims=True)
        eq1 = dm == m1
        dmk = jnp.where(eq1, inf, dm)
        m2 = jnp.min(dmk, axis=1, keepdims=True)
        eq2 = dmk == m2
        eqf = jnp.concatenate([eq1, eq2], axis=0).astype(jnp.float32)
        gaug = jax.lax.dot_general(eqf, paug, (((1,), (0,)), ((), ())),
                                   preferred_element_type=jnp.float32)
        anytie = jnp.max(gaug[:, f:f + 1]) > 1.5

        def fix_ties(_):
            ma, oha, dma = exact_extract(dm)
            mb, ohb, dmb = exact_extract(dma)
            dm_ref[...] = dmb
            gfix = jax.lax.dot_general(
                jnp.concatenate([oha, ohb], axis=0), paug,
                (((1,), (0,)), ((), ())),
                preferred_element_type=jnp.float32)
            return gfix, mb

        def no_ties(_):
            dm_ref[...] = jnp.where(eq2, inf, dmk)
            return gaug, m2

        gaug2, m2c = jax.lax.cond(anytie, fix_ties, no_ties, 0)
        h1 = jnp.tanh(gaug2[:, :f] + cq2)
        s = jax.lax.dot_general(h1, w2, (((1,), (0,)), ((), ())),
                                preferred_element_type=jnp.float32)
        acc = jnp.maximum(acc_ref[...], jnp.where(m1 < inf, s[:q], NEG))
        acc_ref[...] = jnp.maximum(acc, jnp.where(m2c < inf, s[q:], NEG))
        return carry

    jax.lax.fori_loop(0, KNN // 2, pass_body, 0)
    out_ref[0] = jnp.tanh(acc_ref[...] + b2)


def _sa1_body(qcol_ref, prow_ref, s1_ref, wp_ref, w1r_ref, b1_ref, w2_ref,
              b2_ref, out_ref, dm_ref, acc_ref):
    b = pl.program_id(0)
    qx = qcol_ref[0, :, 0:1]
    qy = qcol_ref[0, :, 1:2]
    px = prow_ref[pl.ds(b, 1), :]
    py = prow_ref[pl.ds(BB + b, 1), :]
    s1 = s1_ref[0]
    p_feat = (s1[:, 0:1] * wp_ref[0:1, :] + s1[:, 1:2] * wp_ref[1:2, :]
              + s1[:, 2:3] * wp_ref[2:3, :] + s1[:, 3:4] * wp_ref[3:4, :]
              + s1[:, 4:5] * wp_ref[4:5, :])
    cq = b1_ref[...] - qx * w1r_ref[0:1, :] - qy * w1r_ref[1:2, :]
    _knn_conv(qx, qy, px, py, p_feat, cq, w2_ref[...], b2_ref[...], R1SQ,
              out_ref, dm_ref, acc_ref)


def _sa2_body(qcol_ref, prow_ref, scol_ref, x1_ref, w1x_ref, w1r_ref, b1_ref,
              w2_ref, b2_ref, out_ref, dm_ref, acc_ref):
    b = pl.program_id(0)
    qx = qcol_ref[0, :, 0:1]
    qy = qcol_ref[0, :, 1:2]
    px = prow_ref[pl.ds(b, 1), :]
    py = prow_ref[pl.ds(BB + b, 1), :]
    sx = scol_ref[0, :, 0:1]
    sy = scol_ref[0, :, 1:2]
    p_feat = (jax.lax.dot_general(x1_ref[0], w1x_ref[...],
                                  (((1,), (0,)), ((), ())),
                                  preferred_element_type=jnp.float32)
              + sx * w1r_ref[0:1, :] + sy * w1r_ref[1:2, :])
    cq = b1_ref[...] - qx * w1r_ref[0:1, :] - qy * w1r_ref[1:2, :]
    _knn_conv(qx, qy, px, py, p_feat, cq, w2_ref[...], b2_ref[...], R2SQ,
              out_ref, dm_ref, acc_ref)


def _glob_body(s3_ref, wg0_ref, bg0_ref, wg1_ref, bg1_ref, out_ref):
    h = jnp.tanh(
        jax.lax.dot_general(s3_ref[0], wg0_ref[...], (((1,), (0,)), ((), ())),
                            preferred_element_type=jnp.float32) + bg0_ref[...])
    g = jnp.tanh(
        jax.lax.dot_general(h, wg1_ref[...], (((1,), (0,)), ((), ())),
                            preferred_element_type=jnp.float32) + bg1_ref[...])
    out_ref[0] = jnp.max(g, axis=0, keepdims=True)


def kernel(x, pos, local_params, sa1_params, sa2_params, glob_params):
    (wl0, bl0), (wl1, bl1) = local_params
    (w1_sa1, b1_sa1), (w2_sa1, b2_sa1) = sa1_params
    (w1_sa2, b1_sa2), (w2_sa2, b2_sa2) = sa2_params
    (wg0, bg0), (wg1, bg1) = glob_params

    f1, c1 = w1_sa1.shape[1], w2_sa1.shape[1]
    f2, c2 = w1_sa2.shape[1], w2_sa2.shape[1]

    local_features = pl.pallas_call(
        _local_body,
        grid=(BB,),
        in_specs=[
            pl.BlockSpec((1, MM, 2), lambda b: (b, 0, 0)),
            pl.BlockSpec((2, 64), lambda b: (0, 0)),
            pl.BlockSpec((1, 64), lambda b: (0, 0)),
            pl.BlockSpec((64, 128), lambda b: (0, 0)),
            pl.BlockSpec((1, 128), lambda b: (0, 0)),
        ],
        out_specs=pl.BlockSpec((1, MM, 128), lambda b: (b, 0, 0)),
        out_shape=jax.ShapeDtypeStruct((BB, MM, 128), jnp.float32),
    )(pos, wl0, bl0[None, :], wl1, bl1[None, :])

    pxy = jnp.transpose(pos, (2, 0, 1))
    q1, q2 = pl.pallas_call(
        _fps_body,
        out_shape=(
            jax.ShapeDtypeStruct((2, BB, N1), jnp.float32),
            jax.ShapeDtypeStruct((2, BB, N2), jnp.float32),
        ),
    )(pxy)
    q1col = jnp.transpose(q1, (1, 2, 0))
    q2col = jnp.transpose(q2, (1, 2, 0))
    pxy2 = pxy.reshape(2 * BB, MM)
    q1row2 = q1.reshape(2 * BB, N1)

    s1 = jnp.concatenate([pos, x], axis=-1)
    wp = w1_sa1[0:2] + w1_sa1[5:7]
    wp = jnp.concatenate([wp, w1_sa1[2:5]], axis=0)
    w1r_sa1 = w1_sa1[5:7]
    x1 = pl.pallas_call(
        _sa1_body,
        grid=(BB, N1 // QT1),
        in_specs=[
            pl.BlockSpec((1, QT1, 2), lambda b, t: (b, t, 0)),
            pl.BlockSpec((2 * BB, MM), lambda b, t: (0, 0)),
            pl.BlockSpec((1, MM, 5), lambda b, t: (b, 0, 0)),
            pl.BlockSpec((5, f1), lambda b, t: (0, 0)),
            pl.BlockSpec((2, f1), lambda b, t: (0, 0)),
            pl.BlockSpec((1, f1), lambda b, t: (0, 0)),
            pl.BlockSpec((f1, c1), lambda b, t: (0, 0)),
            pl.BlockSpec((1, c1), lambda b, t: (0, 0)),
        ],
        out_specs=pl.BlockSpec((1, QT1, c1), lambda b, t: (b, t, 0)),
        out_shape=jax.ShapeDtypeStruct((BB, N1, c1), jnp.float32),
        scratch_shapes=[pltpu.VMEM((QT1, MM), jnp.float32),
                        pltpu.VMEM((QT1, c1), jnp.float32)],
    )(q1col, pxy2, s1, wp, w1r_sa1, b1_sa1[None, :], w2_sa1, b2_sa1[None, :])

    w1x_sa2 = w1_sa2[0:128]
    w1r_sa2 = w1_sa2[128:130]
    x2 = pl.pallas_call(
        _sa2_body,
        grid=(BB,),
        in_specs=[
            pl.BlockSpec((1, N2, 2), lambda b: (b, 0, 0)),
            pl.BlockSpec((2 * BB, N1), lambda b: (0, 0)),
            pl.BlockSpec((1, N1, 2), lambda b: (b, 0, 0)),
            pl.BlockSpec((1, N1, 128), lambda b: (b, 0, 0)),
            pl.BlockSpec((128, f2), lambda b: (0, 0)),
            pl.BlockSpec((2, f2), lambda b: (0, 0)),
            pl.BlockSpec((1, f2), lambda b: (0, 0)),
            pl.BlockSpec((f2, c2), lambda b: (0, 0)),
            pl.BlockSpec((1, c2), lambda b: (0, 0)),
        ],
        out_specs=pl.BlockSpec((1, N2, c2), lambda b: (b, 0, 0)),
        out_shape=jax.ShapeDtypeStruct((BB, N2, c2), jnp.float32),
        scratch_shapes=[pltpu.VMEM((N2, N1), jnp.float32),
                        pltpu.VMEM((N2, c2), jnp.float32)],
    )(q2col, q1row2, q1col, x1, w1x_sa2, w1r_sa2, b1_sa2[None, :], w2_sa2,
      b2_sa2[None, :])

    s3 = jnp.concatenate([x2, q2col], axis=-1)
    gf = pl.pallas_call(
        _glob_body,
        grid=(BB,),
        in_specs=[
            pl.BlockSpec((1, N2, 258), lambda b: (b, 0, 0)),
            pl.BlockSpec((258, 256), lambda b: (0, 0)),
            pl.BlockSpec((1, 256), lambda b: (0, 0)),
            pl.BlockSpec((256, 512), lambda b: (0, 0)),
            pl.BlockSpec((1, 512), lambda b: (0, 0)),
        ],
        out_specs=pl.BlockSpec((1, 1, 512), lambda b: (b, 0, 0)),
        out_shape=jax.ShapeDtypeStruct((BB, 1, 512), jnp.float32),
    )(s3, wg0, bg0[None, :], wg1, bg1[None, :])

    return (local_features, gf)


# speedup vs baseline: 1.0784x; 1.0784x over previous
import jax
import jax.numpy as jnp
from jax.experimental import pallas as pl
from jax.experimental.pallas import tpu as pltpu

BB, MM = 4, 4096
N1, N2 = 1024, 256
KNN = 64
R1SQ = 0.2 * 0.2
R2SQ = 0.4 * 0.4
NEG = -1e30
QT1 = 256


def _local_body(pos_ref, wl0_ref, bl0_ref, wl1_ref, bl1_ref, out_ref):
    px = pos_ref[0, :, 0:1]
    py = pos_ref[0, :, 1:2]
    h = jnp.tanh(px * wl0_ref[0:1, :] + py * wl0_ref[1:2, :] + bl0_ref[...])
    out_ref[0] = jnp.tanh(
        jax.lax.dot_general(h, wl1_ref[...], (((1,), (0,)), ((), ())),
                            preferred_element_type=jnp.float32)
        + bl1_ref[...])


def _fps_run(px, py, n):
    b, msrc = px.shape
    iota = jax.lax.broadcasted_iota(jnp.int32, (b, msrc), 1)
    slot_iota = jax.lax.broadcasted_iota(jnp.int32, (b, n), 1)

    qx0 = jnp.where(slot_iota == 0, px[:, 0:1], 0.0)
    qy0 = jnp.where(slot_iota == 0, py[:, 0:1], 0.0)
    mind0 = jnp.full((b, msrc), jnp.inf, dtype=jnp.float32)

    def body(i, st):
        qx, qy, mind, lx, ly = st
        d = (px - lx) ** 2 + (py - ly) ** 2
        mind = jnp.minimum(mind, d)
        mx = jnp.max(mind, axis=1, keepdims=True)
        isel = jnp.min(jnp.where(mind == mx, iota, msrc), axis=1, keepdims=True)
        sel = iota == isel
        nlx = jnp.sum(jnp.where(sel, px, 0.0), axis=1, keepdims=True)
        nly = jnp.sum(jnp.where(sel, py, 0.0), axis=1, keepdims=True)
        slot = slot_iota == (i + 1)
        qx = jnp.where(slot, nlx, qx)
        qy = jnp.where(slot, nly, qy)
        return (qx, qy, mind, nlx, nly)

    qx, qy, _, _, _ = jax.lax.fori_loop(
        0, n - 1, body, (qx0, qy0, mind0, px[:, 0:1], py[:, 0:1]))
    return qx, qy


def _fps_body(pxy_ref, q1_ref, q2_ref):
    px = pxy_ref[0]
    py = pxy_ref[1]
    q1x, q1y = _fps_run(px, py, N1)
    q1_ref[0] = q1x
    q1_ref[1] = q1y
    q2x, q2y = _fps_run(q1x, q1y, N2)
    q2_ref[0] = q2x
    q2_ref[1] = q2y


def _knn_conv(qx, qy, px, py, p_feat, cq, w2, b2, rsq, out_ref, dm_ref,
              acc_ref):
    q, _ = qx.shape
    msrc = px.shape[1]
    inf = jnp.float32(jnp.inf)
    c = w2.shape[1]
    f = w2.shape[0]
    iota = jax.lax.broadcasted_iota(jnp.int32, (q, msrc), 1)
    paug = jnp.concatenate(
        [p_feat, jnp.ones((msrc, 1), jnp.float32)], axis=1)

    d = (qx - px) ** 2 + (qy - py) ** 2
    dm_ref[...] = jnp.where(d <= rsq, d, inf)
    acc_ref[...] = jnp.full((q, c), NEG, jnp.float32)
    cq2 = jnp.concatenate([cq] * 4, axis=0)

    def exact_extract(dm):
        m = jnp.min(dm, axis=1, keepdims=True)
        cand = jnp.where(dm == m, iota, msrc)
        isel = jnp.min(cand, axis=1, keepdims=True)
        oh = cand == isel
        return m, oh.astype(jnp.float32), jnp.where(oh, inf, dm)

    ndeep = 4

    def pass_body(_, carry):
        dm0 = dm_ref[...]
        ms, eqs = [], []
        dm = dm0
        for _i in range(ndeep):
            m = jnp.min(dm, axis=1, keepdims=True)
            eq = dm == m
            ms.append(m)
            eqs.append(eq)
            dm = jnp.where(eq, inf, dm)
        eqf = jnp.concatenate(eqs, axis=0).astype(jnp.float32)
        gaug = jax.lax.dot_general(eqf, paug, (((1,), (0,)), ((), ())),
                                   preferred_element_type=jnp.float32)
        anytie = jnp.max(gaug[:, f:f + 1]) > 1.5

        def fix_ties(_):
            ohs, mfs = [], []
            dmf = dm0
            for _i in range(ndeep):
                mf, oh, dmf = exact_extract(dmf)
                ohs.append(oh)
                mfs.append(mf)
            dm_ref[...] = dmf
            gfix = jax.lax.dot_general(
                jnp.concatenate(ohs, axis=0), paug,
                (((1,), (0,)), ((), ())),
                preferred_element_type=jnp.float32)
            return gfix, jnp.concatenate(mfs, axis=0)

        def no_ties(_):
            dm_ref[...] = dm
            return gaug, jnp.concatenate(ms, axis=0)

        gaug2, mc = jax.lax.cond(anytie, fix_ties, no_ties, 0)
        h1 = jnp.tanh(gaug2[:, :f] + cq2)
        s = jax.lax.dot_general(h1, w2, (((1,), (0,)), ((), ())),
                                preferred_element_type=jnp.float32)
        acc = acc_ref[...]
        for _i in range(ndeep):
            acc = jnp.maximum(
                acc, jnp.where(mc[_i * q:(_i + 1) * q] < inf,
                               s[_i * q:(_i + 1) * q], NEG))
        acc_ref[...] = acc
        return carry

    jax.lax.fori_loop(0, KNN // ndeep, pass_body, 0)
    out_ref[0] = jnp.tanh(acc_ref[...] + b2)


def _sa1_body(qcol_ref, prow_ref, s1_ref, wp_ref, w1r_ref, b1_ref, w2_ref,
              b2_ref, out_ref, dm_ref, acc_ref):
    b = pl.program_id(0)
    qx = qcol_ref[0, :, 0:1]
    qy = qcol_ref[0, :, 1:2]
    px = prow_ref[pl.ds(b, 1), :]
    py = prow_ref[pl.ds(BB + b, 1), :]
    s1 = s1_ref[0]
    p_feat = (s1[:, 0:1] * wp_ref[0:1, :] + s1[:, 1:2] * wp_ref[1:2, :]
              + s1[:, 2:3] * wp_ref[2:3, :] + s1[:, 3:4] * wp_ref[3:4, :]
              + s1[:, 4:5] * wp_ref[4:5, :])
    cq = b1_ref[...] - qx * w1r_ref[0:1, :] - qy * w1r_ref[1:2, :]
    _knn_conv(qx, qy, px, py, p_feat, cq, w2_ref[...], b2_ref[...], R1SQ,
              out_ref, dm_ref, acc_ref)


def _sa2_body(qcol_ref, prow_ref, scol_ref, x1_ref, w1x_ref, w1r_ref, b1_ref,
              w2_ref, b2_ref, out_ref, dm_ref, acc_ref):
    b = pl.program_id(0)
    qx = qcol_ref[0, :, 0:1]
    qy = qcol_ref[0, :, 1:2]
    px = prow_ref[pl.ds(b, 1), :]
    py = prow_ref[pl.ds(BB + b, 1), :]
    sx = scol_ref[0, :, 0:1]
    sy = scol_ref[0, :, 1:2]
    p_feat = (jax.lax.dot_general(x1_ref[0], w1x_ref[...],
                                  (((1,), (0,)), ((), ())),
                                  preferred_element_type=jnp.float32)
              + sx * w1r_ref[0:1, :] + sy * w1r_ref[1:2, :])
    cq = b1_ref[...] - qx * w1r_ref[0:1, :] - qy * w1r_ref[1:2, :]
    _knn_conv(qx, qy, px, py, p_feat, cq, w2_ref[...], b2_ref[...], R2SQ,
              out_ref, dm_ref, acc_ref)


def _glob_body(s3_ref, wg0_ref, bg0_ref, wg1_ref, bg1_ref, out_ref):
    h = jnp.tanh(
        jax.lax.dot_general(s3_ref[0], wg0_ref[...], (((1,), (0,)), ((), ())),
                            preferred_element_type=jnp.float32) + bg0_ref[...])
    g = jnp.tanh(
        jax.lax.dot_general(h, wg1_ref[...], (((1,), (0,)), ((), ())),
                            preferred_element_type=jnp.float32) + bg1_ref[...])
    out_ref[0] = jnp.max(g, axis=0, keepdims=True)


def kernel(x, pos, local_params, sa1_params, sa2_params, glob_params):
    (wl0, bl0), (wl1, bl1) = local_params
    (w1_sa1, b1_sa1), (w2_sa1, b2_sa1) = sa1_params
    (w1_sa2, b1_sa2), (w2_sa2, b2_sa2) = sa2_params
    (wg0, bg0), (wg1, bg1) = glob_params

    f1, c1 = w1_sa1.shape[1], w2_sa1.shape[1]
    f2, c2 = w1_sa2.shape[1], w2_sa2.shape[1]

    local_features = pl.pallas_call(
        _local_body,
        grid=(BB,),
        in_specs=[
            pl.BlockSpec((1, MM, 2), lambda b: (b, 0, 0)),
            pl.BlockSpec((2, 64), lambda b: (0, 0)),
            pl.BlockSpec((1, 64), lambda b: (0, 0)),
            pl.BlockSpec((64, 128), lambda b: (0, 0)),
            pl.BlockSpec((1, 128), lambda b: (0, 0)),
        ],
        out_specs=pl.BlockSpec((1, MM, 128), lambda b: (b, 0, 0)),
        out_shape=jax.ShapeDtypeStruct((BB, MM, 128), jnp.float32),
    )(pos, wl0, bl0[None, :], wl1, bl1[None, :])

    pxy = jnp.transpose(pos, (2, 0, 1))
    q1, q2 = pl.pallas_call(
        _fps_body,
        out_shape=(
            jax.ShapeDtypeStruct((2, BB, N1), jnp.float32),
            jax.ShapeDtypeStruct((2, BB, N2), jnp.float32),
        ),
    )(pxy)
    q1col = jnp.transpose(q1, (1, 2, 0))
    q2col = jnp.transpose(q2, (1, 2, 0))
    pxy2 = pxy.reshape(2 * BB, MM)
    q1row2 = q1.reshape(2 * BB, N1)

    s1 = jnp.concatenate([pos, x], axis=-1)
    wp = w1_sa1[0:2] + w1_sa1[5:7]
    wp = jnp.concatenate([wp, w1_sa1[2:5]], axis=0)
    w1r_sa1 = w1_sa1[5:7]
    x1 = pl.pallas_call(
        _sa1_body,
        grid=(BB, N1 // QT1),
        in_specs=[
            pl.BlockSpec((1, QT1, 2), lambda b, t: (b, t, 0)),
            pl.BlockSpec((2 * BB, MM), lambda b, t: (0, 0)),
            pl.BlockSpec((1, MM, 5), lambda b, t: (b, 0, 0)),
            pl.BlockSpec((5, f1), lambda b, t: (0, 0)),
            pl.BlockSpec((2, f1), lambda b, t: (0, 0)),
            pl.BlockSpec((1, f1), lambda b, t: (0, 0)),
            pl.BlockSpec((f1, c1), lambda b, t: (0, 0)),
            pl.BlockSpec((1, c1), lambda b, t: (0, 0)),
        ],
        out_specs=pl.BlockSpec((1, QT1, c1), lambda b, t: (b, t, 0)),
        out_shape=jax.ShapeDtypeStruct((BB, N1, c1), jnp.float32),
        scratch_shapes=[pltpu.VMEM((QT1, MM), jnp.float32),
                        pltpu.VMEM((QT1, c1), jnp.float32)],
    )(q1col, pxy2, s1, wp, w1r_sa1, b1_sa1[None, :], w2_sa1, b2_sa1[None, :])

    w1x_sa2 = w1_sa2[0:128]
    w1r_sa2 = w1_sa2[128:130]
    x2 = pl.pallas_call(
        _sa2_body,
        grid=(BB,),
        in_specs=[
            pl.BlockSpec((1, N2, 2), lambda b: (b, 0, 0)),
            pl.BlockSpec((2 * BB, N1), lambda b: (0, 0)),
            pl.BlockSpec((1, N1, 2), lambda b: (b, 0, 0)),
            pl.BlockSpec((1, N1, 128), lambda b: (b, 0, 0)),
            pl.BlockSpec((128, f2), lambda b: (0, 0)),
            pl.BlockSpec((2, f2), lambda b: (0, 0)),
            pl.BlockSpec((1, f2), lambda b: (0, 0)),
            pl.BlockSpec((f2, c2), lambda b: (0, 0)),
            pl.BlockSpec((1, c2), lambda b: (0, 0)),
        ],
        out_specs=pl.BlockSpec((1, N2, c2), lambda b: (b, 0, 0)),
        out_shape=jax.ShapeDtypeStruct((BB, N2, c2), jnp.float32),
        scratch_shapes=[pltpu.VMEM((N2, N1), jnp.float32),
                        pltpu.VMEM((N2, c2), jnp.float32)],
    )(q2col, q1row2, q1col, x1, w1x_sa2, w1r_sa2, b1_sa2[None, :], w2_sa2,
      b2_sa2[None, :])

    s3 = jnp.concatenate([x2, q2col], axis=-1)
    gf = pl.pallas_call(
        _glob_body,
        grid=(BB,),
        in_specs=[
            pl.BlockSpec((1, N2, 258), lambda b: (b, 0, 0)),
            pl.BlockSpec((258, 256), lambda b: (0, 0)),
            pl.BlockSpec((1, 256), lambda b: (0, 0)),
            pl.BlockSpec((256, 512), lambda b: (0, 0)),
            pl.BlockSpec((1, 512), lambda b: (0, 0)),
        ],
        out_specs=pl.BlockSpec((1, 1, 512), lambda b: (b, 0, 0)),
        out_shape=jax.ShapeDtypeStruct((BB, 1, 512), jnp.float32),
    )(s3, wg0, bg0[None, :], wg1, bg1[None, :])

    return (local_features, gf)
